# baseline (speedup 1.0000x reference)
import jax
import jax.numpy as jnp
from jax.experimental import pallas as pl
from jax.experimental.pallas import tpu as pltpu

_VMEM_LIMIT = 48 * 1024 * 1024


def _conv_stack_kernel(lhs1_ref, rhs1_ref, sh1_ref, w2_ref, sh2_ref,
                       w3_ref, sh3_ref, w4_ref, sh4_ref, o_ref,
                       zb2_ref, zb3_ref, zb4_ref):
    f32 = jnp.float32

    y1 = jnp.dot(lhs1_ref[0], rhs1_ref[...], preferred_element_type=f32)
    m1 = jnp.maximum(y1[:, 0:64], y1[:, 64:128])
    m1 = jnp.maximum(m1[:, 0:32], m1[:, 32:64])
    p1 = jnp.maximum(m1 + sh1_ref[...], 0.0)
    v1 = p1.reshape(64, 64, 32)
    zb2_ref[0] = jnp.zeros_like(zb2_ref[0])
    zb2_ref[65] = jnp.zeros_like(zb2_ref[65])
    zb2_ref[:, 0:1, 0:32] = jnp.zeros_like(zb2_ref[:, 0:1, 0:32])
    zb2_ref[:, 63:64, 64:96] = jnp.zeros_like(zb2_ref[:, 63:64, 64:96])
    zb2_ref[1:65, 1:65, 0:32] = v1
    zb2_ref[1:65, 0:64, 32:64] = v1
    zb2_ref[1:65, 0:63, 64:96] = v1[:, 1:64]

    y2 = jnp.dot(zb2_ref[0:64].reshape(4608, 96), w2_ref[0],
                 preferred_element_type=f32)
    for kh in range(1, 3):
        y2 = y2 + jnp.dot(zb2_ref[kh:kh + 64].reshape(4608, 96), w2_ref[kh],
                          preferred_element_type=f32)
    p2 = jnp.max(y2.reshape(32, 2, 36, 2, 64), axis=(1, 3))
    p2 = jnp.maximum(p2 + sh2_ref[...], 0.0)
    zb3_ref[0] = jnp.zeros_like(zb3_ref[0])
    zb3_ref[33] = jnp.zeros_like(zb3_ref[33])
    zb3_ref[:, 0:1, 0:64] = jnp.zeros_like(zb3_ref[:, 0:1, 0:64])
    zb3_ref[:, 31:32, 128:192] = jnp.zeros_like(zb3_ref[:, 31:32, 128:192])
    zb3_ref[1:33, 1:33, 0:64] = p2[:, 0:32]
    zb3_ref[1:33, 0:32, 64:128] = p2[:, 0:32]
    zb3_ref[1:33, 0:31, 128:192] = p2[:, 1:32]

    y3 = jnp.dot(zb3_ref[0:32].reshape(1280, 192), w3_ref[0],
                 preferred_element_type=f32)
    for kh in range(1, 3):
        y3 = y3 + jnp.dot(zb3_ref[kh:kh + 32].reshape(1280, 192), w3_ref[kh],
                          preferred_element_type=f32)
    p3 = jnp.max(y3.reshape(16, 2, 20, 2, 128), axis=(1, 3))
    p3 = jnp.maximum(p3 + sh3_ref[...], 0.0)
    zb4_ref[0] = jnp.zeros_like(zb4_ref[0])
    zb4_ref[17] = jnp.zeros_like(zb4_ref[17])
    zb4_ref[:, 0:1, 0:128] = jnp.zeros_like(zb4_ref[:, 0:1, 0:128])
    zb4_ref[:, 15:16, 256:384] = jnp.zeros_like(zb4_ref[:, 15:16, 256:384])
    zb4_ref[1:17, 1:17, 0:128] = p3[:, 0:16]
    zb4_ref[1:17, 0:16, 128:256] = p3[:, 0:16]
    zb4_ref[1:17, 0:15, 256:384] = p3[:, 1:16]

    y4 = jnp.dot(zb4_ref[0:16].reshape(384, 384), w4_ref[0],
                 preferred_element_type=f32)
    for kh in range(1, 3):
        y4 = y4 + jnp.dot(zb4_ref[kh:kh + 16].reshape(384, 384), w4_ref[kh],
                          preferred_element_type=f32)
    p4 = jnp.max(y4.reshape(4, 4, 6, 4, 256), axis=(1, 3))
    p4 = jnp.maximum(p4 + sh4_ref[...], 0.0)[:, 0:4, :]
    o_ref[...] = p4.reshape(1, 1, 4096).astype(jnp.bfloat16)


def _mlp_kernel(x_ref, w1_ref, b1_ref, w2_ref, b2_ref, o_ref):
    h = jnp.dot(x_ref[...], w1_ref[...], preferred_element_type=jnp.float32)
    h = jnp.maximum(h + b1_ref[...], 0.0)
    o_ref[...] = (jnp.dot(h.astype(jnp.bfloat16), w2_ref[...],
                          preferred_element_type=jnp.float32) + b2_ref[...])


def _build_rhs1(w, scale):
    ws = w[:, :, 0, :] * scale[None, None, :]
    r = jnp.zeros((16, 128), jnp.float32)
    for di in range(2):
        for dj in range(2):
            p = di * 2 + dj
            for kh in range(3):
                for kw in range(3):
                    row = (di + kh) * 4 + (dj + kw)
                    r = r.at[row, p * 32:(p + 1) * 32].set(ws[kh, kw])
    return r.astype(jnp.bfloat16)


def kernel(x, conv1_w, conv1_scale, conv1_shift, conv2_w, conv2_scale,
           conv2_shift, conv3_w, conv3_scale, conv3_shift, conv4_w,
           conv4_scale, conv4_shift, fc_w1, fc_b1, fc_w2, fc_b2):
    B = x.shape[0]
    bf16 = jnp.bfloat16

    xp = jnp.pad(x[:, 0], ((0, 0), (1, 1), (1, 1)))
    patches = jnp.stack(
        [xp[:, a:a + 128:2, b:b + 128:2] for a in range(4) for b in range(4)],
        axis=-1)
    lhs1 = patches.reshape(B, 4096, 16).astype(bf16)

    rhs1 = _build_rhs1(conv1_w, conv1_scale)
    sh1 = conv1_shift.reshape(1, 32)
    w2 = (conv2_w * conv2_scale).transpose(0, 1, 2, 3).reshape(3, 96, 64)
    w3 = (conv3_w * conv3_scale).reshape(3, 192, 128)
    w4 = (conv4_w * conv4_scale).reshape(3, 384, 256)
    sh2 = conv2_shift.reshape(1, 64)
    sh3 = conv3_shift.reshape(1, 128)
    sh4 = conv4_shift.reshape(1, 256)

    feats = pl.pallas_call(
        _conv_stack_kernel,
        out_shape=jax.ShapeDtypeStruct((B, 1, 4096), bf16),
        grid=(B,),
        in_specs=[
            pl.BlockSpec((1, 4096, 16), lambda b: (b, 0, 0)),
            pl.BlockSpec((16, 128), lambda b: (0, 0)),
            pl.BlockSpec((1, 32), lambda b: (0, 0)),
            pl.BlockSpec((3, 96, 64), lambda b: (0, 0, 0)),
            pl.BlockSpec((1, 64), lambda b: (0, 0)),
            pl.BlockSpec((3, 192, 128), lambda b: (0, 0, 0)),
            pl.BlockSpec((1, 128), lambda b: (0, 0)),
            pl.BlockSpec((3, 384, 256), lambda b: (0, 0, 0)),
            pl.BlockSpec((1, 256), lambda b: (0, 0)),
        ],
        out_specs=pl.BlockSpec((1, 1, 4096), lambda b: (b, 0, 0)),
        scratch_shapes=[
            pltpu.VMEM((66, 72, 96), jnp.float32),
            pltpu.VMEM((34, 40, 192), jnp.float32),
            pltpu.VMEM((18, 24, 384), jnp.float32),
        ],
        compiler_params=pltpu.CompilerParams(
            dimension_semantics=("parallel",),
            vmem_limit_bytes=_VMEM_LIMIT),
    )(lhs1, rhs1, sh1, w2, sh2, w3, sh3, w4, sh4)
    feats = feats.reshape(B, 4096)

    w1p = (fc_w1.reshape(256, 16, 512).transpose(1, 0, 2)
           .reshape(4096, 512).astype(bf16))
    Bh = B // 2
    out = pl.pallas_call(
        _mlp_kernel,
        out_shape=jax.ShapeDtypeStruct((B, 10), jnp.float32),
        grid=(2,),
        in_specs=[
            pl.BlockSpec((Bh, 4096), lambda i: (i, 0)),
            pl.BlockSpec((4096, 512), lambda i: (0, 0)),
            pl.BlockSpec((1, 512), lambda i: (0, 0)),
            pl.BlockSpec((512, 10), lambda i: (0, 0)),
            pl.BlockSpec((1, 10), lambda i: (0, 0)),
        ],
        out_specs=pl.BlockSpec((Bh, 10), lambda i: (i, 0)),
        compiler_params=pltpu.CompilerParams(
            dimension_semantics=("parallel",),
            vmem_limit_bytes=_VMEM_LIMIT),
    )(feats, w1p, fc_b1.reshape(1, 512), fc_w2.astype(bf16),
      fc_b2.reshape(1, 10))
    return out


# speedup vs baseline: 1.0371x; 1.0371x over previous
import jax
import jax.numpy as jnp
from jax.experimental import pallas as pl
from jax.experimental.pallas import tpu as pltpu

_VMEM_LIMIT = 48 * 1024 * 1024


def _conv_stack_kernel(lhs1_ref, rhs1_ref, sh1_ref, w2_ref, sh2_ref,
                       w3_ref, sh3_ref, w4_ref, sh4_ref, o_ref,
                       zb2_ref, zb3_ref, zb4_ref):
    f32 = jnp.float32

    y1 = jnp.dot(lhs1_ref[0], rhs1_ref[...], preferred_element_type=f32)
    m1 = jnp.maximum(y1[:, 0:64], y1[:, 64:128])
    m1 = jnp.maximum(m1[:, 0:32], m1[:, 32:64])
    p1 = jnp.maximum(m1 + sh1_ref[...], 0.0)
    v1 = p1.reshape(64, 64, 32)
    zb2_ref[0] = jnp.zeros_like(zb2_ref[0])
    zb2_ref[65] = jnp.zeros_like(zb2_ref[65])
    zb2_ref[:, 0:1, 0:32] = jnp.zeros_like(zb2_ref[:, 0:1, 0:32])
    zb2_ref[:, 63:64, 64:96] = jnp.zeros_like(zb2_ref[:, 63:64, 64:96])
    zb2_ref[1:65, 1:65, 0:32] = v1
    zb2_ref[1:65, 0:64, 32:64] = v1
    zb2_ref[1:65, 0:63, 64:96] = v1[:, 1:64]

    y2 = jnp.dot(zb2_ref[0:64].reshape(4608, 96), w2_ref[0],
                 preferred_element_type=f32)
    for kh in range(1, 3):
        y2 = y2 + jnp.dot(zb2_ref[kh:kh + 64].reshape(4608, 96), w2_ref[kh],
                          preferred_element_type=f32)
    p2 = jnp.max(y2.reshape(32, 2, 36, 2, 64), axis=(1, 3))
    p2 = jnp.maximum(p2 + sh2_ref[...], 0.0)
    zb3_ref[0] = jnp.zeros_like(zb3_ref[0])
    zb3_ref[33] = jnp.zeros_like(zb3_ref[33])
    zb3_ref[:, 0:1, 0:64] = jnp.zeros_like(zb3_ref[:, 0:1, 0:64])
    zb3_ref[:, 31:32, 128:192] = jnp.zeros_like(zb3_ref[:, 31:32, 128:192])
    zb3_ref[1:33, 1:33, 0:64] = p2[:, 0:32]
    zb3_ref[1:33, 0:32, 64:128] = p2[:, 0:32]
    zb3_ref[1:33, 0:31, 128:192] = p2[:, 1:32]

    y3 = jnp.dot(zb3_ref[0:32].reshape(1280, 192), w3_ref[0],
                 preferred_element_type=f32)
    for kh in range(1, 3):
        y3 = y3 + jnp.dot(zb3_ref[kh:kh + 32].reshape(1280, 192), w3_ref[kh],
                          preferred_element_type=f32)
    p3 = jnp.max(y3.reshape(16, 2, 20, 2, 128), axis=(1, 3))
    p3 = jnp.maximum(p3 + sh3_ref[...], 0.0)
    zb4_ref[0] = jnp.zeros_like(zb4_ref[0])
    zb4_ref[17] = jnp.zeros_like(zb4_ref[17])
    zb4_ref[:, 0:1, 0:128] = jnp.zeros_like(zb4_ref[:, 0:1, 0:128])
    zb4_ref[:, 15:16, 256:384] = jnp.zeros_like(zb4_ref[:, 15:16, 256:384])
    zb4_ref[1:17, 1:17, 0:128] = p3[:, 0:16]
    zb4_ref[1:17, 0:16, 128:256] = p3[:, 0:16]
    zb4_ref[1:17, 0:15, 256:384] = p3[:, 1:16]

    y4 = jnp.dot(zb4_ref[0:16].reshape(384, 384), w4_ref[0],
                 preferred_element_type=f32)
    for kh in range(1, 3):
        y4 = y4 + jnp.dot(zb4_ref[kh:kh + 16].reshape(384, 384), w4_ref[kh],
                          preferred_element_type=f32)
    p4 = jnp.max(y4.reshape(4, 4, 6, 4, 256), axis=(1, 3))
    p4 = jnp.maximum(p4 + sh4_ref[...], 0.0)[:, 0:4, :]
    o_ref[...] = p4.reshape(1, 1, 4096).astype(jnp.bfloat16)


def _mlp_kernel(x_ref, w1_ref, b1_ref, w2_ref, b2_ref, o_ref):
    h = jnp.dot(x_ref[...], w1_ref[...], preferred_element_type=jnp.float32)
    h = jnp.maximum(h + b1_ref[...], 0.0)
    o_ref[...] = (jnp.dot(h.astype(jnp.bfloat16), w2_ref[...],
                          preferred_element_type=jnp.float32) + b2_ref[...])


def _build_rhs1(w, scale):
    ws = w[:, :, 0, :] * scale[None, None, :]
    r = jnp.zeros((16, 128), jnp.float32)
    for di in range(2):
        for dj in range(2):
            p = di * 2 + dj
            for kh in range(3):
                for kw in range(3):
                    row = (di + kh) * 4 + (dj + kw)
                    r = r.at[row, p * 32:(p + 1) * 32].set(ws[kh, kw])
    return r.astype(jnp.bfloat16)


def kernel(x, conv1_w, conv1_scale, conv1_shift, conv2_w, conv2_scale,
           conv2_shift, conv3_w, conv3_scale, conv3_shift, conv4_w,
           conv4_scale, conv4_shift, fc_w1, fc_b1, fc_w2, fc_b2):
    B = x.shape[0]
    bf16 = jnp.bfloat16

    pp = jnp.pad(x.reshape(B, 64, 2, 64, 2).transpose(0, 2, 4, 1, 3),
                 ((0, 0), (0, 0), (0, 0), (1, 1), (1, 1)))
    d = (-1, 0, 0, 1)
    patches = jnp.stack(
        [pp[:, 1 - (a & 1), 1 - (b & 1),
            1 + d[a]:65 + d[a], 1 + d[b]:65 + d[b]]
         for a in range(4) for b in range(4)],
        axis=-1)
    lhs1 = patches.reshape(B, 4096, 16).astype(bf16)

    rhs1 = _build_rhs1(conv1_w, conv1_scale)
    sh1 = conv1_shift.reshape(1, 32)
    w2 = (conv2_w * conv2_scale).transpose(0, 1, 2, 3).reshape(3, 96, 64)
    w3 = (conv3_w * conv3_scale).reshape(3, 192, 128)
    w4 = (conv4_w * conv4_scale).reshape(3, 384, 256)
    sh2 = conv2_shift.reshape(1, 64)
    sh3 = conv3_shift.reshape(1, 128)
    sh4 = conv4_shift.reshape(1, 256)

    feats = pl.pallas_call(
        _conv_stack_kernel,
        out_shape=jax.ShapeDtypeStruct((B, 1, 4096), bf16),
        grid=(B,),
        in_specs=[
            pl.BlockSpec((1, 4096, 16), lambda b: (b, 0, 0)),
            pl.BlockSpec((16, 128), lambda b: (0, 0)),
            pl.BlockSpec((1, 32), lambda b: (0, 0)),
            pl.BlockSpec((3, 96, 64), lambda b: (0, 0, 0)),
            pl.BlockSpec((1, 64), lambda b: (0, 0)),
            pl.BlockSpec((3, 192, 128), lambda b: (0, 0, 0)),
            pl.BlockSpec((1, 128), lambda b: (0, 0)),
            pl.BlockSpec((3, 384, 256), lambda b: (0, 0, 0)),
            pl.BlockSpec((1, 256), lambda b: (0, 0)),
        ],
        out_specs=pl.BlockSpec((1, 1, 4096), lambda b: (b, 0, 0)),
        scratch_shapes=[
            pltpu.VMEM((66, 72, 96), jnp.float32),
            pltpu.VMEM((34, 40, 192), jnp.float32),
            pltpu.VMEM((18, 24, 384), jnp.float32),
        ],
        compiler_params=pltpu.CompilerParams(
            dimension_semantics=("parallel",),
            vmem_limit_bytes=_VMEM_LIMIT),
    )(lhs1, rhs1, sh1, w2, sh2, w3, sh3, w4, sh4)
    feats = feats.reshape(B, 4096)

    w1p = (fc_w1.reshape(256, 16, 512).transpose(1, 0, 2)
           .reshape(4096, 512).astype(bf16))
    Bh = B // 2
    out = pl.pallas_call(
        _mlp_kernel,
        out_shape=jax.ShapeDtypeStruct((B, 10), jnp.float32),
        grid=(2,),
        in_specs=[
            pl.BlockSpec((Bh, 4096), lambda i: (i, 0)),
            pl.BlockSpec((4096, 512), lambda i: (0, 0)),
            pl.BlockSpec((1, 512), lambda i: (0, 0)),
            pl.BlockSpec((512, 10), lambda i: (0, 0)),
            pl.BlockSpec((1, 10), lambda i: (0, 0)),
        ],
        out_specs=pl.BlockSpec((Bh, 10), lambda i: (i, 0)),
        compiler_params=pltpu.CompilerParams(
            dimension_semantics=("parallel",),
            vmem_limit_bytes=_VMEM_LIMIT),
    )(feats, w1p, fc_b1.reshape(1, 512), fc_w2.astype(bf16),
      fc_b2.reshape(1, 10))
    return out


# speedup vs baseline: 3.6497x; 3.5192x over previous
import jax
import jax.numpy as jnp
from jax.experimental import pallas as pl
from jax.experimental.pallas import tpu as pltpu

_VMEM_LIMIT = 48 * 1024 * 1024


def _conv_stack_kernel(lhs1_ref, rhs1_ref, sh1_ref, w2_ref, sh2_ref,
                       w3_ref, sh3_ref, w4_ref, sh4_ref, o_ref,
                       zb2_ref, zb3_ref, zb4_ref):
    for im in range(2):
        _one_image(im, lhs1_ref, rhs1_ref, sh1_ref, w2_ref, sh2_ref,
                   w3_ref, sh3_ref, w4_ref, sh4_ref, o_ref,
                   zb2_ref, zb3_ref, zb4_ref)


def _one_image(im, lhs1_ref, rhs1_ref, sh1_ref, w2_ref, sh2_ref,
               w3_ref, sh3_ref, w4_ref, sh4_ref, o_ref,
               zb2_ref, zb3_ref, zb4_ref):
    f32 = jnp.float32

    y1 = jax.lax.dot_general(lhs1_ref[im], rhs1_ref[...],
                             dimension_numbers=(((0,), (0,)), ((), ())),
                             preferred_element_type=f32)
    m1 = jnp.maximum(y1[:, 0:64], y1[:, 64:128])
    m1 = jnp.maximum(m1[:, 0:32], m1[:, 32:64])
    p1 = jnp.maximum(m1 + sh1_ref[...], 0.0)
    v1 = p1.reshape(64, 64, 32)
    zb2_ref[im, 0] = jnp.zeros_like(zb2_ref[im, 0])
    zb2_ref[im, 65] = jnp.zeros_like(zb2_ref[im, 65])
    zb2_ref[im, :, 0:1, 0:32] = jnp.zeros_like(zb2_ref[im, :, 0:1, 0:32])
    zb2_ref[im, :, 63:64, 64:96] = jnp.zeros_like(zb2_ref[im, :, 63:64, 64:96])
    zb2_ref[im, 1:65, 1:65, 0:32] = v1
    zb2_ref[im, 1:65, 0:64, 32:64] = v1
    zb2_ref[im, 1:65, 0:63, 64:96] = v1[:, 1:64]

    y2 = jnp.dot(zb2_ref[im, 0:64].reshape(4608, 96), w2_ref[0],
                 preferred_element_type=f32)
    for kh in range(1, 3):
        y2 = y2 + jnp.dot(zb2_ref[im, kh:kh + 64].reshape(4608, 96), w2_ref[kh],
                          preferred_element_type=f32)
    p2 = jnp.max(y2.reshape(32, 2, 36, 2, 64), axis=(1, 3))
    p2 = jnp.maximum(p2 + sh2_ref[...], 0.0)
    zb3_ref[im, 0] = jnp.zeros_like(zb3_ref[im, 0])
    zb3_ref[im, 33] = jnp.zeros_like(zb3_ref[im, 33])
    zb3_ref[im, :, 0:1, 0:64] = jnp.zeros_like(zb3_ref[im, :, 0:1, 0:64])
    zb3_ref[im, :, 31:32, 128:192] = jnp.zeros_like(zb3_ref[im, :, 31:32, 128:192])
    zb3_ref[im, 1:33, 1:33, 0:64] = p2[:, 0:32]
    zb3_ref[im, 1:33, 0:32, 64:128] = p2[:, 0:32]
    zb3_ref[im, 1:33, 0:31, 128:192] = p2[:, 1:32]

    y3 = jnp.dot(zb3_ref[im, 0:32].reshape(1280, 192), w3_ref[0],
                 preferred_element_type=f32)
    for kh in range(1, 3):
        y3 = y3 + jnp.dot(zb3_ref[im, kh:kh + 32].reshape(1280, 192), w3_ref[kh],
                          preferred_element_type=f32)
    p3 = jnp.max(y3.reshape(16, 2, 20, 2, 128), axis=(1, 3))
    p3 = jnp.maximum(p3 + sh3_ref[...], 0.0)
    zb4_ref[im, 0] = jnp.zeros_like(zb4_ref[im, 0])
    zb4_ref[im, 17] = jnp.zeros_like(zb4_ref[im, 17])
    zb4_ref[im, :, 0:1, 0:128] = jnp.zeros_like(zb4_ref[im, :, 0:1, 0:128])
    zb4_ref[im, :, 15:16, 256:384] = jnp.zeros_like(zb4_ref[im, :, 15:16, 256:384])
    zb4_ref[im, 1:17, 1:17, 0:128] = p3[:, 0:16]
    zb4_ref[im, 1:17, 0:16, 128:256] = p3[:, 0:16]
    zb4_ref[im, 1:17, 0:15, 256:384] = p3[:, 1:16]

    y4 = jnp.dot(zb4_ref[im, 0:16].reshape(384, 384), w4_ref[0],
                 preferred_element_type=f32)
    for kh in range(1, 3):
        y4 = y4 + jnp.dot(zb4_ref[im, kh:kh + 16].reshape(384, 384), w4_ref[kh],
                          preferred_element_type=f32)
    p4 = jnp.max(y4.reshape(4, 4, 6, 4, 256), axis=(1, 3))
    p4 = jnp.maximum(p4 + sh4_ref[...], 0.0)[:, 0:4, :]
    o_ref[im] = p4.reshape(1, 4096).astype(jnp.bfloat16)


def _mlp_kernel(x_ref, w1_ref, b1_ref, w2_ref, b2_ref, o_ref):
    h = jnp.dot(x_ref[...], w1_ref[...], preferred_element_type=jnp.float32)
    h = jnp.maximum(h + b1_ref[...], 0.0)
    o_ref[...] = (jnp.dot(h.astype(jnp.bfloat16), w2_ref[...],
                          preferred_element_type=jnp.float32) + b2_ref[...])


def _build_rhs1(w, scale):
    ws = w[:, :, 0, :] * scale[None, None, :]
    r = jnp.zeros((16, 128), jnp.float32)
    for di in range(2):
        for dj in range(2):
            p = di * 2 + dj
            for kh in range(3):
                for kw in range(3):
                    row = (di + kh) * 4 + (dj + kw)
                    r = r.at[row, p * 32:(p + 1) * 32].set(ws[kh, kw])
    return r.astype(jnp.bfloat16)


def kernel(x, conv1_w, conv1_scale, conv1_shift, conv2_w, conv2_scale,
           conv2_shift, conv3_w, conv3_scale, conv3_shift, conv4_w,
           conv4_scale, conv4_shift, fc_w1, fc_b1, fc_w2, fc_b2):
    B = x.shape[0]
    bf16 = jnp.bfloat16

    pp = jnp.pad(x.reshape(B, 64, 2, 64, 2).transpose(0, 2, 4, 1, 3),
                 ((0, 0), (0, 0), (0, 0), (1, 1), (1, 1)))
    d = (-1, 0, 0, 1)
    lhs1 = jnp.stack(
        [pp[:, 1 - (a & 1), 1 - (b & 1),
            1 + d[a]:65 + d[a], 1 + d[b]:65 + d[b]].reshape(B, 4096)
         for a in range(4) for b in range(4)],
        axis=1).astype(bf16)

    rhs1 = _build_rhs1(conv1_w, conv1_scale)
    sh1 = conv1_shift.reshape(1, 32)
    w2 = (conv2_w * conv2_scale).reshape(3, 96, 64)
    w3 = (conv3_w * conv3_scale).reshape(3, 192, 128)
    w4 = (conv4_w * conv4_scale).reshape(3, 384, 256)
    sh2 = conv2_shift.reshape(1, 64)
    sh3 = conv3_shift.reshape(1, 128)
    sh4 = conv4_shift.reshape(1, 256)

    feats = pl.pallas_call(
        _conv_stack_kernel,
        out_shape=jax.ShapeDtypeStruct((B, 1, 4096), bf16),
        grid=(B // 2,),
        in_specs=[
            pl.BlockSpec((2, 16, 4096), lambda b: (b, 0, 0)),
            pl.BlockSpec((16, 128), lambda b: (0, 0)),
            pl.BlockSpec((1, 32), lambda b: (0, 0)),
            pl.BlockSpec((3, 96, 64), lambda b: (0, 0, 0)),
            pl.BlockSpec((1, 64), lambda b: (0, 0)),
            pl.BlockSpec((3, 192, 128), lambda b: (0, 0, 0)),
            pl.BlockSpec((1, 128), lambda b: (0, 0)),
            pl.BlockSpec((3, 384, 256), lambda b: (0, 0, 0)),
            pl.BlockSpec((1, 256), lambda b: (0, 0)),
        ],
        out_specs=pl.BlockSpec((2, 1, 4096), lambda b: (b, 0, 0)),
        scratch_shapes=[
            pltpu.VMEM((2, 66, 72, 96), jnp.float32),
            pltpu.VMEM((2, 34, 40, 192), jnp.float32),
            pltpu.VMEM((2, 18, 24, 384), jnp.float32),
        ],
        compiler_params=pltpu.CompilerParams(
            dimension_semantics=("parallel",),
            vmem_limit_bytes=_VMEM_LIMIT),
    )(lhs1, rhs1, sh1, w2, sh2, w3, sh3, w4, sh4)
    feats = feats.reshape(B, 4096)

    w1p = (fc_w1.reshape(256, 16, 512).transpose(1, 0, 2)
           .reshape(4096, 512).astype(bf16))
    Bh = B // 2
    out = pl.pallas_call(
        _mlp_kernel,
        out_shape=jax.ShapeDtypeStruct((B, 10), jnp.float32),
        grid=(2,),
        in_specs=[
            pl.BlockSpec((Bh, 4096), lambda i: (i, 0)),
            pl.BlockSpec((4096, 512), lambda i: (0, 0)),
            pl.BlockSpec((1, 512), lambda i: (0, 0)),
            pl.BlockSpec((512, 10), lambda i: (0, 0)),
            pl.BlockSpec((1, 10), lambda i: (0, 0)),
        ],
        out_specs=pl.BlockSpec((Bh, 10), lambda i: (i, 0)),
        compiler_params=pltpu.CompilerParams(
            dimension_semantics=("parallel",),
            vmem_limit_bytes=_VMEM_LIMIT),
    )(feats, w1p, fc_b1.reshape(1, 512), fc_w2.astype(bf16),
      fc_b2.reshape(1, 10))
    return out


# speedup vs baseline: 3.8055x; 1.0427x over previous
import jax
import jax.numpy as jnp
from jax.experimental import pallas as pl
from jax.experimental.pallas import tpu as pltpu

_VMEM_LIMIT = 48 * 1024 * 1024


def _conv_stack_kernel(lhs1_ref, rhs1_ref, sh1_ref, w2_ref, sh2_ref,
                       w3_ref, sh3_ref, w4_ref, sh4_ref, o_ref,
                       zb2_ref, zb3_ref, zb4_ref):
    for im in range(4):
        _one_image(im, lhs1_ref, rhs1_ref, sh1_ref, w2_ref, sh2_ref,
                   w3_ref, sh3_ref, w4_ref, sh4_ref, o_ref,
                   zb2_ref, zb3_ref, zb4_ref)


def _one_image(im, lhs1_ref, rhs1_ref, sh1_ref, w2_ref, sh2_ref,
               w3_ref, sh3_ref, w4_ref, sh4_ref, o_ref,
               zb2_ref, zb3_ref, zb4_ref):
    f32 = jnp.float32

    y1 = jax.lax.dot_general(lhs1_ref[im], rhs1_ref[...],
                             dimension_numbers=(((0,), (0,)), ((), ())),
                             preferred_element_type=f32)
    m1 = jnp.maximum(y1[:, 0:64], y1[:, 64:128])
    m1 = jnp.maximum(m1[:, 0:32], m1[:, 32:64])
    p1 = jnp.maximum(m1 + sh1_ref[...], 0.0)
    v1 = p1.reshape(64, 64, 32)
    zb2_ref[im, 0] = jnp.zeros_like(zb2_ref[im, 0])
    zb2_ref[im, 65] = jnp.zeros_like(zb2_ref[im, 65])
    zb2_ref[im, :, 0:1, 0:32] = jnp.zeros_like(zb2_ref[im, :, 0:1, 0:32])
    zb2_ref[im, :, 63:64, 64:96] = jnp.zeros_like(zb2_ref[im, :, 63:64, 64:96])
    zb2_ref[im, 1:65, 1:65, 0:32] = v1
    zb2_ref[im, 1:65, 0:64, 32:64] = v1
    zb2_ref[im, 1:65, 0:63, 64:96] = v1[:, 1:64]

    y2 = jnp.dot(zb2_ref[im, 0:64].reshape(4608, 96), w2_ref[0],
                 preferred_element_type=f32)
    for kh in range(1, 3):
        y2 = y2 + jnp.dot(zb2_ref[im, kh:kh + 64].reshape(4608, 96), w2_ref[kh],
                          preferred_element_type=f32)
    p2 = jnp.max(y2.reshape(32, 2, 36, 2, 64), axis=(1, 3))
    p2 = jnp.maximum(p2 + sh2_ref[...], 0.0)
    zb3_ref[im, 0] = jnp.zeros_like(zb3_ref[im, 0])
    zb3_ref[im, 33] = jnp.zeros_like(zb3_ref[im, 33])
    zb3_ref[im, :, 0:1, 0:64] = jnp.zeros_like(zb3_ref[im, :, 0:1, 0:64])
    zb3_ref[im, :, 31:32, 128:192] = jnp.zeros_like(zb3_ref[im, :, 31:32, 128:192])
    zb3_ref[im, 1:33, 1:33, 0:64] = p2[:, 0:32]
    zb3_ref[im, 1:33, 0:32, 64:128] = p2[:, 0:32]
    zb3_ref[im, 1:33, 0:31, 128:192] = p2[:, 1:32]

    y3 = jnp.dot(zb3_ref[im, 0:32].reshape(1280, 192), w3_ref[0],
                 preferred_element_type=f32)
    for kh in range(1, 3):
        y3 = y3 + jnp.dot(zb3_ref[im, kh:kh + 32].reshape(1280, 192), w3_ref[kh],
                          preferred_element_type=f32)
    p3 = jnp.max(y3.reshape(16, 2, 20, 2, 128), axis=(1, 3))
    p3 = jnp.maximum(p3 + sh3_ref[...], 0.0)
    zb4_ref[im, 0] = jnp.zeros_like(zb4_ref[im, 0])
    zb4_ref[im, 17] = jnp.zeros_like(zb4_ref[im, 17])
    zb4_ref[im, :, 0:1, 0:128] = jnp.zeros_like(zb4_ref[im, :, 0:1, 0:128])
    zb4_ref[im, :, 15:16, 256:384] = jnp.zeros_like(zb4_ref[im, :, 15:16, 256:384])
    zb4_ref[im, 1:17, 1:17, 0:128] = p3[:, 0:16]
    zb4_ref[im, 1:17, 0:16, 128:256] = p3[:, 0:16]
    zb4_ref[im, 1:17, 0:15, 256:384] = p3[:, 1:16]

    y4 = jnp.dot(zb4_ref[im, 0:16].reshape(384, 384), w4_ref[0],
                 preferred_element_type=f32)
    for kh in range(1, 3):
        y4 = y4 + jnp.dot(zb4_ref[im, kh:kh + 16].reshape(384, 384), w4_ref[kh],
                          preferred_element_type=f32)
    p4 = jnp.max(y4.reshape(4, 4, 6, 4, 256), axis=(1, 3))
    p4 = jnp.maximum(p4 + sh4_ref[...], 0.0)[:, 0:4, :]
    o_ref[im] = p4.reshape(1, 4096).astype(jnp.bfloat16)


def _mlp_kernel(x_ref, w1_ref, b1_ref, w2_ref, b2_ref, o_ref):
    h = jnp.dot(x_ref[...], w1_ref[...], preferred_element_type=jnp.float32)
    h = jnp.maximum(h + b1_ref[...], 0.0)
    o_ref[...] = (jnp.dot(h.astype(jnp.bfloat16), w2_ref[...],
                          preferred_element_type=jnp.float32) + b2_ref[...])


def _build_rhs1(w, scale):
    ws = w[:, :, 0, :] * scale[None, None, :]
    r = jnp.zeros((16, 128), jnp.float32)
    for di in range(2):
        for dj in range(2):
            p = di * 2 + dj
            for kh in range(3):
                for kw in range(3):
                    row = (di + kh) * 4 + (dj + kw)
                    r = r.at[row, p * 32:(p + 1) * 32].set(ws[kh, kw])
    return r.astype(jnp.bfloat16)


def kernel(x, conv1_w, conv1_scale, conv1_shift, conv2_w, conv2_scale,
           conv2_shift, conv3_w, conv3_scale, conv3_shift, conv4_w,
           conv4_scale, conv4_shift, fc_w1, fc_b1, fc_w2, fc_b2):
    B = x.shape[0]
    bf16 = jnp.bfloat16

    pp = jnp.pad(x.reshape(B, 64, 2, 64, 2).transpose(0, 2, 4, 1, 3),
                 ((0, 0), (0, 0), (0, 0), (1, 1), (1, 1)))
    d = (-1, 0, 0, 1)
    lhs1 = jnp.stack(
        [pp[:, 1 - (a & 1), 1 - (b & 1),
            1 + d[a]:65 + d[a], 1 + d[b]:65 + d[b]].reshape(B, 4096)
         for a in range(4) for b in range(4)],
        axis=1).astype(bf16)

    rhs1 = _build_rhs1(conv1_w, conv1_scale)
    sh1 = conv1_shift.reshape(1, 32)
    w2 = (conv2_w * conv2_scale).reshape(3, 96, 64)
    w3 = (conv3_w * conv3_scale).reshape(3, 192, 128)
    w4 = (conv4_w * conv4_scale).reshape(3, 384, 256)
    sh2 = conv2_shift.reshape(1, 64)
    sh3 = conv3_shift.reshape(1, 128)
    sh4 = conv4_shift.reshape(1, 256)

    feats = pl.pallas_call(
        _conv_stack_kernel,
        out_shape=jax.ShapeDtypeStruct((B, 1, 4096), bf16),
        grid=(B // 4,),
        in_specs=[
            pl.BlockSpec((4, 16, 4096), lambda b: (b, 0, 0)),
            pl.BlockSpec((16, 128), lambda b: (0, 0)),
            pl.BlockSpec((1, 32), lambda b: (0, 0)),
            pl.BlockSpec((3, 96, 64), lambda b: (0, 0, 0)),
            pl.BlockSpec((1, 64), lambda b: (0, 0)),
            pl.BlockSpec((3, 192, 128), lambda b: (0, 0, 0)),
            pl.BlockSpec((1, 128), lambda b: (0, 0)),
            pl.BlockSpec((3, 384, 256), lambda b: (0, 0, 0)),
            pl.BlockSpec((1, 256), lambda b: (0, 0)),
        ],
        out_specs=pl.BlockSpec((4, 1, 4096), lambda b: (b, 0, 0)),
        scratch_shapes=[
            pltpu.VMEM((4, 66, 72, 96), jnp.float32),
            pltpu.VMEM((4, 34, 40, 192), jnp.float32),
            pltpu.VMEM((4, 18, 24, 384), jnp.float32),
        ],
        compiler_params=pltpu.CompilerParams(
            dimension_semantics=("parallel",),
            vmem_limit_bytes=_VMEM_LIMIT),
    )(lhs1, rhs1, sh1, w2, sh2, w3, sh3, w4, sh4)
    feats = feats.reshape(B, 4096)

    w1p = (fc_w1.reshape(256, 16, 512).transpose(1, 0, 2)
           .reshape(4096, 512).astype(bf16))
    Bh = B // 2
    out = pl.pallas_call(
        _mlp_kernel,
        out_shape=jax.ShapeDtypeStruct((B, 10), jnp.float32),
        grid=(2,),
        in_specs=[
            pl.BlockSpec((Bh, 4096), lambda i: (i, 0)),
            pl.BlockSpec((4096, 512), lambda i: (0, 0)),
            pl.BlockSpec((1, 512), lambda i: (0, 0)),
            pl.BlockSpec((512, 10), lambda i: (0, 0)),
            pl.BlockSpec((1, 10), lambda i: (0, 0)),
        ],
        out_specs=pl.BlockSpec((Bh, 10), lambda i: (i, 0)),
        compiler_params=pltpu.CompilerParams(
            dimension_semantics=("parallel",),
            vmem_limit_bytes=_VMEM_LIMIT),
    )(feats, w1p, fc_b1.reshape(1, 512), fc_w2.astype(bf16),
      fc_b2.reshape(1, 10))
    return out
